# baseline (speedup 1.0000x reference)
import jax
import jax.numpy as jnp
from jax.experimental import pallas as pl
from jax.experimental.pallas import tpu as pltpu

_T, _B, _V = 16, 32, 100000
_G = 2
_NS = _T // _G


def _err_rate_kernel(x_ref, tgt_ref, out_ref):
    s = pl.program_id(0)

    pltpu.prng_seed(jnp.int32(0x12345678) + s)
    bits = pltpu.bitcast(pltpu.prng_random_bits((_G, _B, _V)), jnp.int32)
    r = (bits & jnp.int32(0x7FFFFFFF)).astype(jnp.float32)
    neg_ln_u = jnp.float32(31.0 * 0.6931471805599453) - jnp.log(r)
    val = x_ref[...] - jnp.log(neg_ln_u)
    sample = jnp.argmax(val, axis=2).astype(jnp.int32)

    tgt = tgt_ref[0]
    msk = (tgt != -1).astype(jnp.float32)
    err = (sample != tgt).astype(jnp.float32)
    num = jnp.sum(err * msk, axis=1)
    den = jnp.sum(msk, axis=1)
    out_ref[0, 0, 0] = jnp.sum(num / jnp.maximum(den, 1.0)) * (1.0 / _T)


def kernel(sequence_of_logits, sequence_of_targets):
    tgt3 = sequence_of_targets.reshape(_NS, _G, _B)
    out = pl.pallas_call(
        _err_rate_kernel,
        grid=(_NS,),
        in_specs=[
            pl.BlockSpec((_G, _B, _V), lambda s: (s, 0, 0)),
            pl.BlockSpec((1, _G, _B), lambda s: (s, 0, 0)),
        ],
        out_specs=pl.BlockSpec(
            block_shape=(1, 1, 1),
            index_map=lambda s: (s, 0, 0),
            memory_space=pltpu.SMEM,
        ),
        out_shape=jax.ShapeDtypeStruct((_NS, 1, 1), jnp.float32),
        compiler_params=pltpu.CompilerParams(
            dimension_semantics=("arbitrary",),
        ),
    )(sequence_of_logits, tgt3)
    return jnp.sum(out[:, 0, 0])


# speedup vs baseline: 1.0153x; 1.0153x over previous
import jax
import jax.numpy as jnp
from jax.experimental import pallas as pl
from jax.experimental.pallas import tpu as pltpu

_T, _B, _V = 16, 32, 100000
_G = 2
_NS = _T // _G


def _err_rate_kernel(x_ref, tgt_ref, out_ref):
    s = pl.program_id(0)

    pltpu.prng_seed(jnp.int32(0x12345678) + s)
    bits = pltpu.bitcast(pltpu.prng_random_bits((_G, _B, _V)), jnp.int32)
    r = (bits & jnp.int32(0x7FFFFFFF)).astype(jnp.float32)
    neg_ln_u = jnp.float32(31.0 * 0.6931471805599453) - jnp.log(r)
    val = x_ref[...] - jnp.log(neg_ln_u)
    sample = jnp.argmax(val, axis=2).astype(jnp.int32)

    tgt = tgt_ref[0]
    msk = (tgt != -1).astype(jnp.float32)
    err = (sample != tgt).astype(jnp.float32)
    num = jnp.sum(err * msk, axis=1)
    den = jnp.sum(msk, axis=1)
    part = jnp.sum(num / jnp.maximum(den, 1.0)) * (1.0 / _T)

    @pl.when(s == 0)
    def _init():
        out_ref[0, 0] = 0.0

    out_ref[0, 0] += part


def kernel(sequence_of_logits, sequence_of_targets):
    tgt3 = sequence_of_targets.reshape(_NS, _G, _B)
    out = pl.pallas_call(
        _err_rate_kernel,
        grid=(_NS,),
        in_specs=[
            pl.BlockSpec((_G, _B, _V), lambda s: (s, 0, 0)),
            pl.BlockSpec((1, _G, _B), lambda s: (s, 0, 0)),
        ],
        out_specs=pl.BlockSpec(
            block_shape=(1, 1),
            index_map=lambda s: (0, 0),
            memory_space=pltpu.SMEM,
        ),
        out_shape=jax.ShapeDtypeStruct((1, 1), jnp.float32),
        compiler_params=pltpu.CompilerParams(
            dimension_semantics=("arbitrary",),
        ),
    )(sequence_of_logits, tgt3)
    return out[0, 0]
